# baseline (speedup 1.0000x reference)
import functools

import jax
import jax.numpy as jnp
from jax import lax
from jax.experimental import pallas as pl
from jax.experimental.pallas import tpu as pltpu
from jax.experimental.pallas import tpu_sc as plsc

NUM_CORES = 2
NUM_SUBCORES = 16
LANES = 16
NW = NUM_CORES * NUM_SUBCORES


def _build_sc_call(seq: int, table_rows: int, hid: int, chunk: int):
    max_len = (table_rows + 1) // 2
    rows_per_w = seq // NW
    n_chunks = seq // chunk
    win = rows_per_w + chunk - 1
    win_pad = ((win + LANES - 1) // LANES) * LANES
    groups = win_pad // LANES

    mesh = plsc.VectorSubcoreMesh(
        core_axis_name="c", subcore_axis_name="s",
        num_cores=NUM_CORES, num_subcores=NUM_SUBCORES)

    @functools.partial(
        pl.kernel,
        out_type=jax.ShapeDtypeStruct((seq, seq, hid), jnp.float32),
        mesh=mesh,
        scratch_types=[
            pltpu.VMEM((win_pad,), jnp.int32),
            pltpu.VMEM((win_pad,), jnp.int32),
            pltpu.VMEM((win_pad, hid), jnp.float32),
            pltpu.VMEM((win_pad, hid), jnp.float32),
            pltpu.SemaphoreType.DMA,
            pltpu.SemaphoreType.DMA,
            pltpu.SemaphoreType.DMA,
            pltpu.SemaphoreType.DMA,
        ],
    )
    def sc_gather(table_hbm, out_hbm, idx0, idx1, wb0, wb1, gs0, gs1, ws0, ws1):
        idx, wbuf, gsem, wsem = (idx0, idx1), (wb0, wb1), (gs0, gs1), (ws0, ws1)
        wid = lax.axis_index("s") * NUM_CORES + lax.axis_index("c")
        lane = lax.iota(jnp.int32, LANES)
        i0 = wid * rows_per_w

        def start_gather(b):
            p = b % 2
            j0 = b * chunk
            hi = i0 - j0 + (max_len - 1) + (rows_per_w - 1)
            for g in range(groups):
                idx[p][pl.ds(g * LANES, LANES)] = jnp.maximum(
                    (hi - g * LANES) - lane, 0)
            pltpu.make_async_copy(table_hbm.at[idx[p]], wbuf[p], gsem[p]).start()

        def row_copy(b, di):
            p = b % 2
            return pltpu.make_async_copy(
                wbuf[p].at[pl.ds(rows_per_w - 1 - di, chunk), :],
                out_hbm.at[i0 + di, pl.ds(b * chunk, chunk), :],
                wsem[p])

        start_gather(0)
        for b in range(n_chunks):
            p = b % 2
            pltpu.make_async_copy(table_hbm.at[idx[p]], wbuf[p], gsem[p]).wait()
            for di in range(rows_per_w):
                row_copy(b, di).start()
            if b > 0:
                for di in range(rows_per_w):
                    row_copy(b - 1, di).wait()
            if b + 1 < n_chunks:
                start_gather(b + 1)
        for di in range(rows_per_w):
            row_copy(n_chunks - 1, di).wait()

    return sc_gather


def kernel(x, relative_positions):
    seq = x.shape[1]
    table_rows, hid = relative_positions.shape
    call = _build_sc_call(seq, table_rows, hid, chunk=256)
    return call(relative_positions)
